# Initial kernel scaffold; baseline (speedup 1.0000x reference)
import jax
import jax.numpy as jnp
from jax.experimental import pallas as pl


def kernel(observation, A, C, init_noise, step_noise, resample_u):
    raise NotImplementedError("write your pallas kernel here")


# speedup vs baseline: 4.0117x; 4.0117x over previous
import jax
import jax.numpy as jnp
from jax.experimental import pallas as pl
from jax.experimental.pallas import tpu as pltpu

D_X = 4
D_Y = 4
N_PART = 512
BATCH = 8
T_EXT = 16
SIG_X = 0.5
SIG_Y = 0.5

_F32 = jnp.float32
_HI = jax.lax.Precision.HIGHEST
_BIG = 3.0e38


def _resample_gather_body(cumt_ref, u_ref, x_ref, out_ref):
    n_lane = jax.lax.broadcasted_iota(jnp.int32, (1, N_PART), 1).astype(_F32)
    row_i = jax.lax.broadcasted_iota(jnp.int32, (N_PART, BATCH), 0)
    cumt = cumt_ref[...]
    cumt = jnp.where(row_i == N_PART - 1, _BIG, cumt)
    cumt_prev = jnp.concatenate(
        [jnp.full((1, BATCH), -_BIG, _F32), cumt[:N_PART - 1, :]], axis=0)
    for b in range(BATCH):
        p_row = (n_lane + u_ref[b]) * (1.0 / N_PART)
        onehot = jnp.where(
            (cumt[:, b:b + 1] >= p_row) & (cumt_prev[:, b:b + 1] < p_row),
            1.0, 0.0).astype(_F32)
        out_ref[b] = jax.lax.dot_general(
            x_ref[b], onehot, (((1,), (0,)), ((), ())), precision=_HI)


def _resample_gather(logw, u, x):
    w = jnp.exp(logw)
    cumt = jnp.cumsum(w, axis=1).T
    x_dmaj = x.transpose(0, 2, 1)
    out = pl.pallas_call(
        _resample_gather_body,
        in_specs=[
            pl.BlockSpec(memory_space=pltpu.VMEM),
            pl.BlockSpec(memory_space=pltpu.SMEM),
            pl.BlockSpec(memory_space=pltpu.VMEM),
        ],
        out_specs=pl.BlockSpec(memory_space=pltpu.VMEM),
        out_shape=jax.ShapeDtypeStruct((BATCH, D_X, N_PART), _F32),
    )(cumt, u, x_dmaj)
    return out.transpose(0, 2, 1)


def _log_obs_density(x, y, C):
    mean = jnp.einsum('od,bnd->bno', C, x)
    diff = (y[:, None, :] - mean) / SIG_Y
    return (-0.5 * jnp.sum(diff * diff, axis=-1)
            - 0.5 * D_Y * jnp.log(2.0 * jnp.pi * SIG_Y ** 2))


def kernel(observation, A, C, init_noise, step_noise, resample_u):
    x = init_noise
    lw_un = _log_obs_density(x, observation[0], C)
    logw = lw_un - jax.nn.logsumexp(lw_un, axis=1, keepdims=True)
    outputs = [jnp.einsum('bn,bnd->bd', jnp.exp(logw), x)]
    for t in range(1, T_EXT + 1):
        x_res = _resample_gather(jax.lax.stop_gradient(logw),
                                 resample_u[t - 1], x)
        x_new = jnp.einsum('od,bnd->bno', A, x_res) + SIG_X * step_noise[t - 1]
        lw_un = _log_obs_density(x_new, observation[t], C)
        logw = lw_un - jax.nn.logsumexp(lw_un, axis=1, keepdims=True)
        x = x_new
        outputs.append(jnp.einsum('bn,bnd->bd', jnp.exp(logw), x))
    return jnp.stack(outputs, axis=0)


# speedup vs baseline: 5.5595x; 1.3858x over previous
import jax
import jax.numpy as jnp
from jax.experimental import pallas as pl
from jax.experimental.pallas import tpu as pltpu

D_X = 4
D_Y = 4
N_PART = 512
BATCH = 8
T_EXT = 16
SIG_X = 0.5
SIG_Y = 0.5

_F32 = jnp.float32
_CHUNK = 128
_NCHUNK = N_PART // _CHUNK


def _gather_chunks(planes, idx):
    wi = jax.lax.bitwise_and(idx, _CHUNK - 1)
    ch = jax.lax.shift_right_logical(idx, 7)
    out = jnp.take_along_axis(planes[0], wi, axis=1)
    for c in range(1, _NCHUNK):
        out = jnp.where(ch == c, jnp.take_along_axis(planes[c], wi, axis=1),
                        out)
    return out


def _resample_gather_body(cum_ref, u_ref, x_ref, out_ref, prev_out_ref):
    n_f = jax.lax.broadcasted_iota(jnp.int32, (BATCH, N_PART), 1).astype(_F32)
    p = (n_f + u_ref[...]) * (1.0 / N_PART)
    cum_planes = [cum_ref[:, c * _CHUNK:(c + 1) * _CHUNK]
                  for c in range(_NCHUNK)]
    lo = jnp.zeros((BATCH, N_PART), jnp.int32)
    hi = jnp.full((BATCH, N_PART), N_PART, jnp.int32)
    for _ in range(10):
        mid = jnp.minimum(jax.lax.shift_right_logical(lo + hi, 1), N_PART - 1)
        cv = _gather_chunks(cum_planes, mid)
        pred = cv < p
        lo = jnp.where(pred, mid + 1, lo)
        hi = jnp.where(pred, hi, mid)
    idx = jnp.minimum(lo, N_PART - 1)
    cum = cum_ref[...]
    w = cum - jnp.concatenate(
        [jnp.zeros((BATCH, 1), _F32), cum[:, :N_PART - 1]], axis=1)
    for d in range(D_X):
        x_d = x_ref[d]
        planes = [x_d[:, c * _CHUNK:(c + 1) * _CHUNK] for c in range(_NCHUNK)]
        out_ref[d] = _gather_chunks(planes, idx)
        prev_out_ref[:, d:d + 1] = jnp.sum(w * x_d, axis=1, keepdims=True)


def _resample_gather(logw, u, x):
    w = jnp.exp(logw)
    cum = jnp.cumsum(w, axis=1)
    x_dmaj = x.transpose(2, 0, 1)
    x_res, prev_out = pl.pallas_call(
        _resample_gather_body,
        in_specs=[
            pl.BlockSpec(memory_space=pltpu.VMEM),
            pl.BlockSpec(memory_space=pltpu.VMEM),
            pl.BlockSpec(memory_space=pltpu.VMEM),
        ],
        out_specs=[
            pl.BlockSpec(memory_space=pltpu.VMEM),
            pl.BlockSpec(memory_space=pltpu.VMEM),
        ],
        out_shape=[
            jax.ShapeDtypeStruct((D_X, BATCH, N_PART), _F32),
            jax.ShapeDtypeStruct((BATCH, D_X), _F32),
        ],
    )(cum, u[:, None], x_dmaj)
    return x_res.transpose(1, 2, 0), prev_out


def _log_obs_density(x, y, C):
    mean = jnp.einsum('od,bnd->bno', C, x)
    diff = (y[:, None, :] - mean) / SIG_Y
    return (-0.5 * jnp.sum(diff * diff, axis=-1)
            - 0.5 * D_Y * jnp.log(2.0 * jnp.pi * SIG_Y ** 2))


def kernel(observation, A, C, init_noise, step_noise, resample_u):
    x = init_noise
    lw_un = _log_obs_density(x, observation[0], C)
    logw = lw_un - jax.nn.logsumexp(lw_un, axis=1, keepdims=True)
    outputs = [None] * (T_EXT + 1)
    for t in range(1, T_EXT + 1):
        x_res, prev_out = _resample_gather(jax.lax.stop_gradient(logw),
                                           resample_u[t - 1], x)
        outputs[t - 1] = prev_out
        x_new = jnp.einsum('od,bnd->bno', A, x_res) + SIG_X * step_noise[t - 1]
        lw_un = _log_obs_density(x_new, observation[t], C)
        logw = lw_un - jax.nn.logsumexp(lw_un, axis=1, keepdims=True)
        x = x_new
    outputs[T_EXT] = jnp.einsum('bn,bnd->bd', jnp.exp(logw), x)
    return jnp.stack(outputs, axis=0)


# speedup vs baseline: 6.2210x; 1.1190x over previous
import jax
import jax.numpy as jnp
from jax.experimental import pallas as pl
from jax.experimental.pallas import tpu as pltpu

D_X = 4
D_Y = 4
N_PART = 512
BATCH = 8
T_EXT = 16
SIG_X = 0.5
SIG_Y = 0.5

_F32 = jnp.float32
_CHUNK = 128
_NCHUNK = N_PART // _CHUNK


def _gather_chunks(planes, idx):
    wi = jax.lax.bitwise_and(idx, _CHUNK - 1)
    ch = jax.lax.shift_right_logical(idx, 7)
    out = jnp.take_along_axis(planes[0], wi, axis=1)
    for c in range(1, _NCHUNK):
        out = jnp.where(ch == c, jnp.take_along_axis(planes[c], wi, axis=1),
                        out)
    return out


def _resample_gather_body(cum_ref, u_ref, x_ref, out_ref):
    n_f = jax.lax.broadcasted_iota(jnp.int32, (BATCH, N_PART), 1).astype(_F32)
    p = (n_f + u_ref[...]) * (1.0 / N_PART)
    cum_planes = [cum_ref[:, c * _CHUNK:(c + 1) * _CHUNK]
                  for c in range(_NCHUNK)]
    lo = jnp.zeros((BATCH, N_PART), jnp.int32)
    hi = jnp.full((BATCH, N_PART), N_PART, jnp.int32)
    for _ in range(10):
        mid = jnp.minimum(jax.lax.shift_right_logical(lo + hi, 1), N_PART - 1)
        cv = _gather_chunks(cum_planes, mid)
        pred = cv < p
        lo = jnp.where(pred, mid + 1, lo)
        hi = jnp.where(pred, hi, mid)
    idx = jnp.minimum(lo, N_PART - 1)
    for d in range(D_X):
        x_d = x_ref[d]
        planes = [x_d[:, c * _CHUNK:(c + 1) * _CHUNK] for c in range(_NCHUNK)]
        out_ref[d] = _gather_chunks(planes, idx)


def _resample_gather(logw, u, x):
    w = jnp.exp(logw)
    cum = jnp.cumsum(w, axis=1)
    x_dmaj = x.transpose(2, 0, 1)
    x_res = pl.pallas_call(
        _resample_gather_body,
        in_specs=[
            pl.BlockSpec(memory_space=pltpu.VMEM),
            pl.BlockSpec(memory_space=pltpu.VMEM),
            pl.BlockSpec(memory_space=pltpu.VMEM),
        ],
        out_specs=pl.BlockSpec(memory_space=pltpu.VMEM),
        out_shape=jax.ShapeDtypeStruct((D_X, BATCH, N_PART), _F32),
    )(cum, u[:, None], x_dmaj)
    return x_res.transpose(1, 2, 0)


def _log_obs_density(x, y, C):
    mean = jnp.einsum('od,bnd->bno', C, x)
    diff = (y[:, None, :] - mean) / SIG_Y
    return (-0.5 * jnp.sum(diff * diff, axis=-1)
            - 0.5 * D_Y * jnp.log(2.0 * jnp.pi * SIG_Y ** 2))


def kernel(observation, A, C, init_noise, step_noise, resample_u):
    x = init_noise
    lw_un = _log_obs_density(x, observation[0], C)
    logw = lw_un - jax.nn.logsumexp(lw_un, axis=1, keepdims=True)
    outputs = [None] * (T_EXT + 1)
    for t in range(1, T_EXT + 1):
        outputs[t - 1] = jnp.einsum('bn,bnd->bd', jnp.exp(logw), x)
        x_res = _resample_gather(jax.lax.stop_gradient(logw),
                                 resample_u[t - 1], x)
        x_new = jnp.einsum('od,bnd->bno', A, x_res) + SIG_X * step_noise[t - 1]
        lw_un = _log_obs_density(x_new, observation[t], C)
        logw = lw_un - jax.nn.logsumexp(lw_un, axis=1, keepdims=True)
        x = x_new
    outputs[T_EXT] = jnp.einsum('bn,bnd->bd', jnp.exp(logw), x)
    return jnp.stack(outputs, axis=0)
